# baseline (speedup 1.0000x reference)
import dataclasses
import functools

import jax
import jax.numpy as jnp
from jax.experimental import pallas as pl
from jax.experimental.pallas import tpu as pltpu
from jax.experimental.pallas import tpu_sc as plsc

B = 8
C = 64
N = 100000
R = 32
V = R * R * R

P = 1024
NCHUNK = 98
NPAD = NCHUNK * P
TAIL = N - 97 * P
LANES = 16


def _compute_norm_and_idx(coords):
    mean = jnp.mean(coords, axis=2, keepdims=True)
    cc = coords - mean
    maxn = jnp.max(
        jnp.linalg.norm(cc, axis=1, keepdims=True), axis=2, keepdims=True
    )
    nc = cc / (2.0 * maxn) + 0.5
    ncr = jnp.clip(nc * float(R), 0.0, float(R - 1))
    vox = jnp.round(ncr).astype(jnp.int32)
    flat = (vox[:, 0] * R + vox[:, 1]) * R + vox[:, 2]
    padded = jnp.pad(flat, ((0, 0), (0, NPAD - N)))
    return ncr, padded.reshape(B, NPAD // 128, 128)


def _sc_scatter_kernel(
    feat_hbm, tail_hbm, idx_hbm, out_hbm, acc, idxv, fb, idxv2, fb2, sem0, sem1
):
    wid = jax.lax.axis_index("s") * 2 + jax.lax.axis_index("c")
    row = 8 * (wid // 4)
    s0 = (2 * wid) % 8
    s1 = s0 + 1
    c0 = row + s0
    c1 = c0 + 1
    ones = jnp.full((LANES,), 1.0, dtype=jnp.float32)

    def start_chunk(b, k, npts, ib, fbuf, sem):
        pltpu.async_copy(idx_hbm.at[b, pl.ds(k * 8, 8), :], ib, sem)
        pltpu.async_copy(
            feat_hbm.at[b, pl.ds(row, 8), pl.ds(k * P, npts)],
            fbuf.at[:, pl.ds(0, npts)], sem,
        )

    def wait_chunk(npts, ib, fbuf, sem):
        pltpu.make_async_copy(idx_hbm.at[0, pl.ds(0, 8), :], ib, sem).wait()
        pltpu.make_async_copy(
            feat_hbm.at[0, pl.ds(0, 8), pl.ds(0, npts)],
            fbuf.at[:, pl.ds(0, npts)], sem,
        ).wait()

    ch0 = jnp.zeros((LANES,), jnp.int32)
    ch1 = jnp.full((LANES,), 1, jnp.int32)
    ch2 = jnp.full((LANES,), 2, jnp.int32)

    def process(npts, ib, fbuf):
        @plsc.parallel_loop(0, npts, step=LANES, unroll=4)
        def _group(j):
            iv = ib.at[j // 128, pl.ds(j % 128, LANES)][...]
            iz = iv & (R - 1)
            iy = (iv >> 5) & (R - 1)
            ix = iv >> 10
            x0 = fbuf.at[s0, pl.ds(j, LANES)][...]
            x1 = fbuf.at[s1, pl.ds(j, LANES)][...]
            plsc.addupdate_scatter(acc, [ch0, ix, iy, iz], x0)
            plsc.addupdate_scatter(acc, [ch1, ix, iy, iz], x1)
            plsc.addupdate_scatter(acc, [ch2, ix, iy, iz], ones)

    @pl.loop(0, B)
    def _batch(b):
        start_chunk(b, 0, P, idxv, fb, sem0)

        @pl.loop(0, 3)
        def _zc(ci):
            @pl.loop(0, R)
            def _zx(x):
                @plsc.parallel_loop(0, R, unroll=4)
                def _zy(y):
                    zv = jnp.zeros((LANES,), jnp.float32)
                    acc.at[ci, x, y, pl.ds(0, LANES)][...] = zv
                    acc.at[ci, x, y, pl.ds(LANES, LANES)][...] = zv

        @pl.loop(0, 96, step=2)
        def _pair(k):
            start_chunk(b, k + 1, P, idxv2, fb2, sem1)
            wait_chunk(P, idxv, fb, sem0)
            process(P, idxv, fb)
            start_chunk(b, k + 2, P, idxv, fb, sem0)
            wait_chunk(P, idxv2, fb2, sem1)
            process(P, idxv2, fb2)

        pltpu.async_copy(idx_hbm.at[b, pl.ds(97 * 8, 8), :], idxv2, sem1)
        pltpu.async_copy(
            tail_hbm.at[b, pl.ds(row, 8), :], fb2.at[:, pl.ds(0, 768)], sem1
        )
        wait_chunk(P, idxv, fb, sem0)
        process(P, idxv, fb)
        pltpu.make_async_copy(idx_hbm.at[0, pl.ds(0, 8), :], idxv2, sem1).wait()
        pltpu.make_async_copy(
            tail_hbm.at[0, pl.ds(0, 8), :], fb2.at[:, pl.ds(0, 768)], sem1
        ).wait()
        process(TAIL, idxv2, fb2)

        @pl.loop(0, R)
        def _dx(x):
            @plsc.parallel_loop(0, R, unroll=2)
            def _dy(y):
                for z0 in (0, LANES):
                    zs = pl.ds(z0, LANES)
                    cnt = jnp.maximum(acc.at[2, x, y, zs][...], 1.0)
                    acc.at[0, x, y, zs][...] = acc.at[0, x, y, zs][...] / cnt
                    acc.at[1, x, y, zs][...] = acc.at[1, x, y, zs][...] / cnt

        pltpu.sync_copy(acc.at[0], out_hbm.at[b, c0])
        pltpu.sync_copy(acc.at[1], out_hbm.at[b, c1])


def _sc_scatter(features, idx):
    mesh = plsc.VectorSubcoreMesh(core_axis_name="c", subcore_axis_name="s")
    cp = pltpu.CompilerParams()
    if "needs_layout_passes" in pltpu.CompilerParams.__dataclass_fields__:
        cp = dataclasses.replace(cp, needs_layout_passes=False)
    if "use_tc_tiling_on_sc" in pltpu.CompilerParams.__dataclass_fields__:
        cp = dataclasses.replace(cp, use_tc_tiling_on_sc=False)
    fn = functools.partial(
        pl.kernel,
        compiler_params=cp,
        out_type=jax.ShapeDtypeStruct((B, C, R, R, R), jnp.float32),
        mesh=mesh,
        scratch_types=[
            pltpu.VMEM((3, R, R, R), jnp.float32),
            pltpu.VMEM((8, 128), jnp.int32),
            pltpu.VMEM((8, P), jnp.float32),
            pltpu.VMEM((8, 128), jnp.int32),
            pltpu.VMEM((8, P), jnp.float32),
            pltpu.SemaphoreType.DMA,
            pltpu.SemaphoreType.DMA,
        ],
    )(_sc_scatter_kernel)
    tail = jnp.pad(features[:, :, 97 * P:], ((0, 0), (0, 0), (0, 96)))
    return fn(features, tail, idx)


def kernel(features, coords):
    norm_coords, idx = _compute_norm_and_idx(coords)
    out = _sc_scatter(features, idx)
    return out, norm_coords


# speedup vs baseline: 1.3060x; 1.3060x over previous
import dataclasses
import functools

import jax
import jax.numpy as jnp
from jax.experimental import pallas as pl
from jax.experimental.pallas import tpu as pltpu
from jax.experimental.pallas import tpu_sc as plsc

B = 8
C = 64
N = 100000
R = 32
V = R * R * R

P = 1024
NCHUNK = 98
NPAD = NCHUNK * P
TAIL = N - 97 * P
LANES = 16


def _compute_norm_and_idx(coords):
    mean = jnp.mean(coords, axis=2, keepdims=True)
    cc = coords - mean
    maxn = jnp.max(
        jnp.linalg.norm(cc, axis=1, keepdims=True), axis=2, keepdims=True
    )
    nc = cc / (2.0 * maxn) + 0.5
    ncr = jnp.clip(nc * float(R), 0.0, float(R - 1))
    vox = jnp.round(ncr).astype(jnp.int32)
    flat = (vox[:, 0] * R + vox[:, 1]) * R + vox[:, 2]
    padded = jnp.pad(flat, ((0, 0), (0, NPAD - N)))
    return ncr, padded.reshape(B, NPAD // 128, 128)


def _sc_scatter_kernel(
    feat_hbm, tail_hbm, idx_hbm, out_hbm, acc, idxv, fb, idxv2, fb2, sem0, sem1
):
    wid = jax.lax.axis_index("s") * 2 + jax.lax.axis_index("c")
    row = 8 * (wid // 4)
    s0 = (2 * wid) % 8
    s1 = s0 + 1
    c0 = row + s0
    c1 = c0 + 1
    ones = jnp.full((LANES,), 1.0, dtype=jnp.float32)

    def start_chunk(b, k, npts, ib, fbuf, sem):
        pltpu.async_copy(idx_hbm.at[b, pl.ds(k * 8, 8), :], ib, sem)
        pltpu.async_copy(
            feat_hbm.at[b, pl.ds(row, 8), pl.ds(k * P, npts)],
            fbuf.at[:, pl.ds(0, npts)], sem,
        )

    def wait_chunk(npts, ib, fbuf, sem):
        pltpu.make_async_copy(idx_hbm.at[0, pl.ds(0, 8), :], ib, sem).wait()
        pltpu.make_async_copy(
            feat_hbm.at[0, pl.ds(0, 8), pl.ds(0, npts)],
            fbuf.at[:, pl.ds(0, npts)], sem,
        ).wait()

    def process(npts, ib, fbuf):
        @plsc.parallel_loop(0, npts, step=LANES, unroll=4)
        def _group(j):
            iv = ib.at[j // 128, pl.ds(j % 128, LANES)][...]
            x0 = fbuf.at[s0, pl.ds(j, LANES)][...]
            x1 = fbuf.at[s1, pl.ds(j, LANES)][...]
            plsc.addupdate_scatter(acc, [iv], x0)
            plsc.addupdate_scatter(acc, [iv + V], x1)
            plsc.addupdate_scatter(acc, [iv + 2 * V], ones)

    @pl.loop(0, B)
    def _batch(b):
        start_chunk(b, 0, P, idxv, fb, sem0)

        @plsc.parallel_loop(0, 3 * V, step=LANES, unroll=8)
        def _zero(i):
            acc.at[pl.ds(i, LANES)][...] = jnp.zeros((LANES,), jnp.float32)

        @pl.loop(0, 96, step=2)
        def _pair(k):
            start_chunk(b, k + 1, P, idxv2, fb2, sem1)
            wait_chunk(P, idxv, fb, sem0)
            process(P, idxv, fb)
            start_chunk(b, k + 2, P, idxv, fb, sem0)
            wait_chunk(P, idxv2, fb2, sem1)
            process(P, idxv2, fb2)

        pltpu.async_copy(idx_hbm.at[b, pl.ds(97 * 8, 8), :], idxv2, sem1)
        pltpu.async_copy(
            tail_hbm.at[b, pl.ds(row, 8), :], fb2.at[:, pl.ds(0, 768)], sem1
        )
        wait_chunk(P, idxv, fb, sem0)
        process(P, idxv, fb)
        pltpu.make_async_copy(idx_hbm.at[0, pl.ds(0, 8), :], idxv2, sem1).wait()
        pltpu.make_async_copy(
            tail_hbm.at[0, pl.ds(0, 8), :], fb2.at[:, pl.ds(0, 768)], sem1
        ).wait()
        process(TAIL, idxv2, fb2)

        @plsc.parallel_loop(0, V, step=LANES, unroll=4)
        def _div(j):
            cnt = jnp.maximum(acc.at[pl.ds(2 * V + j, LANES)][...], 1.0)
            acc.at[pl.ds(j, LANES)][...] = acc.at[pl.ds(j, LANES)][...] / cnt
            acc.at[pl.ds(V + j, LANES)][...] = (
                acc.at[pl.ds(V + j, LANES)][...] / cnt
            )

        pltpu.sync_copy(
            acc.at[pl.ds(0, V)], out_hbm.at[pl.ds((b * C + c0) * V, V)]
        )
        pltpu.sync_copy(
            acc.at[pl.ds(V, V)], out_hbm.at[pl.ds((b * C + c1) * V, V)]
        )


def _sc_scatter(features, idx):
    mesh = plsc.VectorSubcoreMesh(core_axis_name="c", subcore_axis_name="s")
    cp = pltpu.CompilerParams()
    if "needs_layout_passes" in pltpu.CompilerParams.__dataclass_fields__:
        cp = dataclasses.replace(cp, needs_layout_passes=False)
    fn = functools.partial(
        pl.kernel,
        compiler_params=cp,
        out_type=jax.ShapeDtypeStruct((B * C * V,), jnp.float32),
        mesh=mesh,
        scratch_types=[
            pltpu.VMEM((3 * V,), jnp.float32),
            pltpu.VMEM((8, 128), jnp.int32),
            pltpu.VMEM((8, P), jnp.float32),
            pltpu.VMEM((8, 128), jnp.int32),
            pltpu.VMEM((8, P), jnp.float32),
            pltpu.SemaphoreType.DMA,
            pltpu.SemaphoreType.DMA,
        ],
    )(_sc_scatter_kernel)
    tail = jnp.pad(features[:, :, 97 * P:], ((0, 0), (0, 0), (0, 96)))
    return fn(features, tail, idx)


def kernel(features, coords):
    norm_coords, idx = _compute_norm_and_idx(coords)
    sums = _sc_scatter(features, idx)
    return sums.reshape(B, C, R, R, R), norm_coords
